# baseline (speedup 1.0000x reference)
import functools

import jax
import jax.numpy as jnp
import numpy as np
from jax import lax
from jax.experimental import pallas as pl
from jax.experimental.pallas import tpu as pltpu
from jax.experimental.pallas import tpu_sc as plsc

B = 16
D = 1024
VOCAB = 100000
NEG = -1e9
BN = 2048
NBLK = (VOCAB + BN - 1) // BN
VPAD = NBLK * BN
NCH = VPAD // 16
NHIST = 8192
HCH = NHIST // 16
BIGI = np.int32(2147483647)

_KEY_LO = np.int32(np.int64(-2147483648)
                   - np.int64(np.float32(-3.0e38).view(np.int32))
                   - 1)


def _key16(v):
    b = lax.bitcast_convert_type(v, jnp.int32)
    return jnp.where(b >= 0, b, jnp.int32(-2147483648) - b - 1)


def _unkey16(k):
    b = jnp.where(k >= 0, k, jnp.int32(-2147483648) - k - 1)
    return lax.bitcast_convert_type(b, jnp.float32)


def _matmul_kernel(hidden_ref, emb_ref, temp_ref, logits_ref,
                   m_ref, z_ref, arg_ref, m_s, z_s, arg_s):
    j = pl.program_id(0)

    @pl.when(j == 0)
    def _init():
        m_s[...] = jnp.full_like(m_s, -jnp.inf)
        z_s[...] = jnp.zeros_like(z_s)
        arg_s[...] = jnp.zeros_like(arg_s)

    logits = jax.lax.dot_general(
        hidden_ref[...], emb_ref[...], (((1,), (1,)), ((), ())),
        preferred_element_type=jnp.float32)
    logits = logits / temp_ref[...]

    col = j * BN + jax.lax.broadcasted_iota(jnp.int32, (B, BN), 1)
    lw = jnp.where(col < VOCAB, logits, -jnp.inf)
    logits_ref[...] = lw

    bm = jnp.max(lw, axis=1, keepdims=True)
    barg = jnp.min(jnp.where(lw == bm, col, BIGI), axis=1, keepdims=True)

    m_old = m_s[...]
    m_new = jnp.maximum(m_old, bm)
    z_s[...] = (z_s[...] * jnp.exp(m_old - m_new)
                + jnp.sum(jnp.exp(lw - m_new), axis=1, keepdims=True))
    arg_s[...] = jnp.where(bm > m_old, barg, arg_s[...])
    m_s[...] = m_new

    @pl.when(j == NBLK - 1)
    def _fin():
        m_ref[...] = m_s[...]
        z_ref[...] = z_s[...]
        arg_ref[...] = arg_s[...]


def _sc_select_body(vflat, m_hbm, z_hbm, tp_hbm, tk_hbm, out_hbm,
                    buf, hcnt, hmass, m16, z16, tp16, tk16, io16):
    cid = lax.axis_index("c")
    sid = lax.axis_index("s")
    h = sid % 2
    row = 8 * cid + sid // 2

    @pl.when(h == 0)
    def _work():
        iota = lax.iota(jnp.int32, 16)
        zf16 = jnp.zeros((16,), jnp.float32)

        pltpu.sync_copy(vflat.at[pl.ds(row * VPAD, VPAD)], buf)
        pltpu.sync_copy(m_hbm, m16)
        pltpu.sync_copy(z_hbm, z16)
        pltpu.sync_copy(tp_hbm, tp16)
        pltpu.sync_copy(tk_hbm, tk16)

        ridx = jnp.full((16,), row, jnp.int32)
        m_b = plsc.load_gather(m16, [ridx])
        budget_b = plsc.load_gather(tp16, [ridx]) * plsc.load_gather(z16, [ridx])
        topk_b = plsc.load_gather(tk16, [ridx])

        def clr(i, _):
            hcnt[pl.ds(i * 16, 16)] = zf16
            hmass[pl.ds(i * 16, 16)] = zf16
            return 0
        lax.fori_loop(0, HCH, clr, 0)

        ones = jnp.ones((16,), jnp.float32)

        def hist(i, _):
            v = buf[pl.ds(i * 16, 16)]
            idx = 4095 - (_key16(v) >> 19)
            e = jnp.exp(v - m_b)
            plsc.addupdate_scatter(hcnt, [idx], ones)
            plsc.addupdate_scatter(hmass, [idx], e)
            return 0
        lax.fori_loop(0, NCH, hist, 0)

        def scan(i, carry):
            cc, cm, fb = carry
            c = hcnt[pl.ds(i * 16, 16)]
            mm = hmass[pl.ds(i * 16, 16)]
            ic = plsc.cumsum(c)
            im = plsc.cumsum(mm)
            a_c = cc + (ic - c)
            a_m = cm + (im - mm)
            bad = jnp.logical_not((a_m <= budget_b) & (a_c < topk_b))
            idx16 = i * 16 + iota
            cand = jnp.where(bad, idx16, BIGI)
            fb = jnp.minimum(fb, zf16.astype(jnp.int32)
                             + lax.reduce_min(cand, axes=(0,)))
            cc = cc + lax.reduce_max(ic, axes=(0,))
            cm = cm + lax.reduce_max(im, axes=(0,))
            return cc, cm, fb

        _, _, fbv = lax.fori_loop(
            0, HCH, scan,
            (zf16, zf16, jnp.full((16,), BIGI, jnp.int32)))
        bc_v = fbv - 1

        def base(i, carry):
            ac, am = carry
            c = hcnt[pl.ds(i * 16, 16)]
            mm = hmass[pl.ds(i * 16, 16)]
            idx16 = i * 16 + iota
            sel = idx16 < bc_v
            return ac + jnp.where(sel, c, 0.0), am + jnp.where(sel, mm, 0.0)
        acv, amv = lax.fori_loop(0, HCH, base, (zf16, zf16))
        base_c = zf16 + lax.reduce_sum(acv, axes=(0,))
        base_m = zf16 + lax.reduce_sum(amv, axes=(0,))

        def cpk(i, w):
            v = buf[pl.ds(i * 16, 16)]
            idx = 4095 - (_key16(v) >> 19)
            mk = idx == bc_v
            plsc.store_compressed(buf.at[pl.ds(w, 16)], v, mask=mk)
            nm = lax.reduce_sum(jnp.where(mk, 1.0, 0.0), axes=(0,))
            return w + nm.astype(jnp.int32)
        nloc = lax.fori_loop(0, NCH, cpk, jnp.int32(0))
        nch_c = (nloc + 15) // 16
        nloc_v = zf16.astype(jnp.int32) + nloc

        kbase = (4095 - bc_v) << 19
        lo0 = kbase - 1
        hi0 = kbase + jnp.int32(1 << 19) - 1

        def bis(_, carry):
            lo, hi = carry
            mid = lo + ((hi - lo) >> 1)
            def acc(i, c2):
                pc, pm = c2
                v = buf[pl.ds(i * 16, 16)]
                ok = ((i * 16 + iota) < nloc_v) & (_key16(v) > mid)
                pc = pc + jnp.where(ok, 1.0, 0.0)
                pm = pm + jnp.where(ok, jnp.exp(v - m_b), 0.0)
                return pc, pm
            pc, pm = lax.fori_loop(0, nch_c, acc, (zf16, zf16))
            cnt = base_c + lax.reduce_sum(pc, axes=(0,))
            mass = base_m + lax.reduce_sum(pm, axes=(0,))
            good = (mass <= budget_b) & (cnt < topk_b)
            lo = jnp.where(good, lo, mid)
            hi = jnp.where(good, mid, hi)
            return lo, hi

        lo, _ = lax.fori_loop(0, 19, bis, (lo0, hi0))
        io16[...] = _unkey16(lo)
        pltpu.sync_copy(io16, out_hbm.at[pl.ds(row * 16, 16)])


def _sc_select(vflat, m, z, tp, tkf):
    mesh = plsc.VectorSubcoreMesh(core_axis_name="c", subcore_axis_name="s",
                                  num_cores=2, num_subcores=16)
    kern = functools.partial(
        pl.kernel,
        out_type=jax.ShapeDtypeStruct((B * 16,), jnp.float32),
        mesh=mesh,
        compiler_params=pltpu.CompilerParams(needs_layout_passes=False),
        scratch_types=[
            pltpu.VMEM((VPAD,), jnp.float32),
            pltpu.VMEM((NHIST,), jnp.float32),
            pltpu.VMEM((NHIST,), jnp.float32),
            pltpu.VMEM((16,), jnp.float32),
            pltpu.VMEM((16,), jnp.float32),
            pltpu.VMEM((16,), jnp.float32),
            pltpu.VMEM((16,), jnp.float32),
            pltpu.VMEM((16,), jnp.float32),
        ],
    )(_sc_select_body)
    return kern(vflat, m, z, tp, tkf)


def _emit_kernel(v_ref, m_ref, t_ref, probs_ref, logp_ref):
    m = m_ref[...]
    t_lo = t_ref[...]
    v = v_ref[:, :VOCAB]
    keep = v > t_lo
    e = jnp.exp(v - m)
    ek = jnp.where(keep, e, 0.0)
    zk = jnp.sum(ek, axis=1, keepdims=True)
    probs_ref[...] = ek * (1.0 / zk)
    logp_ref[...] = (jnp.where(keep, v, NEG) - m) - jnp.log(zk)


@jax.jit
def kernel(hidden_states, embedding, temperatures, top_ps, top_ks):
    temp = temperatures.reshape(B, 1)
    tp = top_ps.reshape(B, 1)
    tk = top_ks.astype(jnp.float32).reshape(B, 1)

    logits, m, z, arg = pl.pallas_call(
        _matmul_kernel,
        grid=(NBLK,),
        in_specs=[
            pl.BlockSpec((B, D), lambda j: (0, 0)),
            pl.BlockSpec((BN, D), lambda j: (j, 0)),
            pl.BlockSpec((B, 1), lambda j: (0, 0)),
        ],
        out_specs=[
            pl.BlockSpec((B, BN), lambda j: (0, j)),
            pl.BlockSpec((B, 1), lambda j: (0, 0)),
            pl.BlockSpec((B, 1), lambda j: (0, 0)),
            pl.BlockSpec((B, 1), lambda j: (0, 0)),
        ],
        out_shape=[
            jax.ShapeDtypeStruct((B, VPAD), jnp.float32),
            jax.ShapeDtypeStruct((B, 1), jnp.float32),
            jax.ShapeDtypeStruct((B, 1), jnp.float32),
            jax.ShapeDtypeStruct((B, 1), jnp.int32),
        ],
        scratch_shapes=[
            pltpu.VMEM((B, 1), jnp.float32),
            pltpu.VMEM((B, 1), jnp.float32),
            pltpu.VMEM((B, 1), jnp.int32),
        ],
    )(hidden_states, embedding, temp)

    vflat = logits.reshape(-1)
    tvec = _sc_select(vflat, m.reshape(-1), z.reshape(-1),
                      top_ps, top_ks.astype(jnp.float32))
    t_lo = tvec.reshape(B, 16)[:, :1]

    probs, logp = pl.pallas_call(
        _emit_kernel,
        out_shape=[
            jax.ShapeDtypeStruct((B, VOCAB), jnp.float32),
            jax.ShapeDtypeStruct((B, VOCAB), jnp.float32),
        ],
        compiler_params=pltpu.CompilerParams(
            vmem_limit_bytes=100 * 1024 * 1024),
    )(logits, m, t_lo)

    return arg[:, 0], probs, logp


# speedup vs baseline: 1.6536x; 1.6536x over previous
import jax
import jax.numpy as jnp
import numpy as np
from jax.experimental import pallas as pl
from jax.experimental.pallas import tpu as pltpu

B = 16
D = 1024
VOCAB = 100000
NEG = -1e9
BN = 2048
NBLK = (VOCAB + BN - 1) // BN
VPAD = NBLK * BN

_KEY_LO = np.int32(np.int64(-2147483648)
                   - np.int64(np.float32(-3.0e38).view(np.int32))
                   - 1)


def _f32_to_key(x):
    b = jax.lax.bitcast_convert_type(x, jnp.int32)
    return jnp.where(b >= 0, b, jnp.int32(-2147483648) - b - 1)


def _key_to_f32(k):
    b = jnp.where(k >= 0, k, jnp.int32(-2147483648) - k - 1)
    return jax.lax.bitcast_convert_type(b, jnp.float32)


def _fused_kernel(hidden_ref, emb_ref, temp_ref, tp_ref, tk_ref,
                  arg_ref, probs_ref, logp_ref,
                  v_s, m_s, z_s, arg_s):
    j = pl.program_id(0)

    @pl.when(j == 0)
    def _init():
        m_s[...] = jnp.full_like(m_s, -jnp.inf)
        z_s[...] = jnp.zeros_like(z_s)
        arg_s[...] = jnp.zeros_like(arg_s)

    logits = jax.lax.dot_general(
        hidden_ref[...], emb_ref[...], (((1,), (1,)), ((), ())),
        preferred_element_type=jnp.float32)
    logits = logits / temp_ref[...]

    col = j * BN + jax.lax.broadcasted_iota(jnp.int32, (B, BN), 1)
    lw = jnp.where(col < VOCAB, logits, -jnp.inf)
    v_s[:, pl.ds(j * BN, BN)] = lw

    bm = jnp.max(lw, axis=1, keepdims=True)
    barg = jnp.min(jnp.where(lw == bm, col, jnp.int32(2147483647)),
                   axis=1, keepdims=True)

    m_old = m_s[...]
    m_new = jnp.maximum(m_old, bm)
    z_s[...] = (z_s[...] * jnp.exp(m_old - m_new)
                + jnp.sum(jnp.exp(lw - m_new), axis=1, keepdims=True))
    arg_s[...] = jnp.where(bm > m_old, barg, arg_s[...])
    m_s[...] = m_new

    @pl.when(j == NBLK - 1)
    def _select_emit():
        m = m_s[...]
        probs_ref[...] = jnp.exp(v_s[:, :VOCAB] - m)

        budget = tp_ref[...] * z_s[...]
        topk = tk_ref[...]

        lo0 = jnp.full((B, 1), _KEY_LO, jnp.int32)
        hi0 = _f32_to_key(m)

        def body(_, carry):
            lo, hi = carry
            step = jnp.maximum((((hi >> 1) - (lo >> 1)) // 3) << 1, 1)
            t1 = lo + step
            t2 = t1 + step
            v = v_s[:, :VOCAB]
            e = probs_ref[...]
            mask1 = v > _key_to_f32(t1)
            mask2 = v > _key_to_f32(t2)
            cnt1 = jnp.sum(jnp.where(mask1, 1.0, 0.0), axis=1, keepdims=True)
            cnt2 = jnp.sum(jnp.where(mask2, 1.0, 0.0), axis=1, keepdims=True)
            mass1 = jnp.sum(jnp.where(mask1, e, 0.0), axis=1, keepdims=True)
            mass2 = jnp.sum(jnp.where(mask2, e, 0.0), axis=1, keepdims=True)
            g1 = (mass1 <= budget) & (cnt1 < topk)
            g2 = (mass2 <= budget) & (cnt2 < topk)
            lo = jnp.where(g1, lo, jnp.where(g2, t1, t2))
            hi = jnp.where(g1, t1, jnp.where(g2, t2, hi))
            return lo, hi

        lo, _ = jax.lax.fori_loop(0, 24, body, (lo0, hi0))
        t_lo = _key_to_f32(lo)

        v = v_s[:, :VOCAB]
        keep = v > t_lo
        ek = jnp.where(keep, probs_ref[...], 0.0)
        zk = jnp.sum(ek, axis=1, keepdims=True)
        arg_ref[...] = arg_s[...]
        probs_ref[...] = ek * (1.0 / zk)
        logp_ref[...] = (jnp.where(keep, v, NEG) - m) - jnp.log(zk)


@jax.jit
def kernel(hidden_states, embedding, temperatures, top_ps, top_ks):
    temp = temperatures.reshape(B, 1)
    tp = top_ps.reshape(B, 1)
    tk = top_ks.astype(jnp.float32).reshape(B, 1)

    arg, probs, logp = pl.pallas_call(
        _fused_kernel,
        grid=(NBLK,),
        in_specs=[
            pl.BlockSpec((B, D), lambda j: (0, 0)),
            pl.BlockSpec((BN, D), lambda j: (j, 0)),
            pl.BlockSpec((B, 1), lambda j: (0, 0)),
            pl.BlockSpec((B, 1), lambda j: (0, 0)),
            pl.BlockSpec((B, 1), lambda j: (0, 0)),
        ],
        out_specs=[
            pl.BlockSpec((B, 1), lambda j: (0, 0)),
            pl.BlockSpec((B, VOCAB), lambda j: (0, 0)),
            pl.BlockSpec((B, VOCAB), lambda j: (0, 0)),
        ],
        out_shape=[
            jax.ShapeDtypeStruct((B, 1), jnp.int32),
            jax.ShapeDtypeStruct((B, VOCAB), jnp.float32),
            jax.ShapeDtypeStruct((B, VOCAB), jnp.float32),
        ],
        scratch_shapes=[
            pltpu.VMEM((B, VPAD), jnp.float32),
            pltpu.VMEM((B, 1), jnp.float32),
            pltpu.VMEM((B, 1), jnp.float32),
            pltpu.VMEM((B, 1), jnp.int32),
        ],
        compiler_params=pltpu.CompilerParams(
            vmem_limit_bytes=100 * 1024 * 1024),
    )(hidden_states, embedding, temp, tp, tk)

    return arg[:, 0], probs, logp


# speedup vs baseline: 1.7315x; 1.0471x over previous
import jax
import jax.numpy as jnp
import numpy as np
from jax.experimental import pallas as pl
from jax.experimental.pallas import tpu as pltpu

B = 16
D = 1024
VOCAB = 100000
NEG = -1e9
BN = 2048
NBLK = (VOCAB + BN - 1) // BN
VPAD = NBLK * BN

_KEY_LO = np.int32(np.int64(-2147483648)
                   - np.int64(np.float32(-3.0e38).view(np.int32))
                   - 1)


def _f32_to_key(x):
    b = jax.lax.bitcast_convert_type(x, jnp.int32)
    return jnp.where(b >= 0, b, jnp.int32(-2147483648) - b - 1)


def _key_to_f32(k):
    b = jnp.where(k >= 0, k, jnp.int32(-2147483648) - k - 1)
    return jax.lax.bitcast_convert_type(b, jnp.float32)


def _fused_kernel(hidden_ref, emb_ref, temp_ref, tp_ref, tk_ref,
                  arg_ref, probs_ref, logp_ref,
                  v_s, m_s, z_s, arg_s):
    j = pl.program_id(0)

    @pl.when(j == 0)
    def _init():
        m_s[...] = jnp.full_like(m_s, -jnp.inf)
        z_s[...] = jnp.zeros_like(z_s)
        arg_s[...] = jnp.zeros_like(arg_s)

    logits = jax.lax.dot_general(
        hidden_ref[...], emb_ref[...], (((1,), (1,)), ((), ())),
        preferred_element_type=jnp.float32)
    logits = logits / temp_ref[...]

    col = j * BN + jax.lax.broadcasted_iota(jnp.int32, (B, BN), 1)
    lw = jnp.where(col < VOCAB, logits, -jnp.inf)
    v_s[:, pl.ds(j * BN, BN)] = lw

    bm = jnp.max(lw, axis=1, keepdims=True)
    barg = jnp.min(jnp.where(lw == bm, col, jnp.int32(2147483647)),
                   axis=1, keepdims=True)

    m_old = m_s[...]
    m_new = jnp.maximum(m_old, bm)
    z_s[...] = (z_s[...] * jnp.exp(m_old - m_new)
                + jnp.sum(jnp.exp(lw - m_new), axis=1, keepdims=True))
    arg_s[...] = jnp.where(bm > m_old, barg, arg_s[...])
    m_s[...] = m_new

    @pl.when(j == NBLK - 1)
    def _select_emit():
        m = m_s[...]
        probs_ref[...] = jnp.exp(v_s[:, :VOCAB] - m)

        budget = tp_ref[...] * z_s[...]
        topk = tk_ref[...]

        lo0 = jnp.full((B, 1), _KEY_LO, jnp.int32)
        hi0 = _f32_to_key(m)
        clo0 = jnp.full((B, 1), 1.0e9, jnp.float32)
        chi0 = jnp.zeros((B, 1), jnp.float32)

        def cond(carry):
            it, lo, hi, clo, chi = carry
            return (it < 34) & jnp.any(clo != chi)

        def body(carry):
            it, lo, hi, clo, chi = carry
            mid = (lo >> 1) + (hi >> 1) + (lo & hi & 1)
            tau = _key_to_f32(mid)
            mask = v_s[:, :VOCAB] > tau
            cnt = jnp.sum(jnp.where(mask, 1.0, 0.0), axis=1, keepdims=True)
            mass = jnp.sum(jnp.where(mask, probs_ref[...], 0.0), axis=1,
                           keepdims=True)
            good = (mass <= budget) & (cnt < topk)
            lo = jnp.where(good, lo, mid)
            hi = jnp.where(good, mid, hi)
            clo = jnp.where(good, clo, cnt)
            chi = jnp.where(good, cnt, chi)
            return it + 1, lo, hi, clo, chi

        _, lo, _, _, _ = jax.lax.while_loop(
            cond, body, (jnp.int32(0), lo0, hi0, clo0, chi0))
        t_lo = _key_to_f32(lo)

        v = v_s[:, :VOCAB]
        keep = v > t_lo
        ek = jnp.where(keep, probs_ref[...], 0.0)
        zk = jnp.sum(ek, axis=1, keepdims=True)
        arg_ref[...] = arg_s[...]
        probs_ref[...] = ek * (1.0 / zk)
        logp_ref[...] = (jnp.where(keep, v, NEG) - m) - jnp.log(zk)


@jax.jit
def kernel(hidden_states, embedding, temperatures, top_ps, top_ks):
    temp = temperatures.reshape(B, 1)
    tp = top_ps.reshape(B, 1)
    tk = top_ks.astype(jnp.float32).reshape(B, 1)

    arg, probs, logp = pl.pallas_call(
        _fused_kernel,
        grid=(NBLK,),
        in_specs=[
            pl.BlockSpec((B, D), lambda j: (0, 0)),
            pl.BlockSpec((BN, D), lambda j: (j, 0)),
            pl.BlockSpec((B, 1), lambda j: (0, 0)),
            pl.BlockSpec((B, 1), lambda j: (0, 0)),
            pl.BlockSpec((B, 1), lambda j: (0, 0)),
        ],
        out_specs=[
            pl.BlockSpec((B, 1), lambda j: (0, 0)),
            pl.BlockSpec((B, VOCAB), lambda j: (0, 0)),
            pl.BlockSpec((B, VOCAB), lambda j: (0, 0)),
        ],
        out_shape=[
            jax.ShapeDtypeStruct((B, 1), jnp.int32),
            jax.ShapeDtypeStruct((B, VOCAB), jnp.float32),
            jax.ShapeDtypeStruct((B, VOCAB), jnp.float32),
        ],
        scratch_shapes=[
            pltpu.VMEM((B, VPAD), jnp.float32),
            pltpu.VMEM((B, 1), jnp.float32),
            pltpu.VMEM((B, 1), jnp.float32),
            pltpu.VMEM((B, 1), jnp.int32),
        ],
        compiler_params=pltpu.CompilerParams(
            vmem_limit_bytes=100 * 1024 * 1024),
    )(hidden_states, embedding, temp, tp, tk)

    return arg[:, 0], probs, logp


# speedup vs baseline: 1.8358x; 1.0603x over previous
import jax
import jax.numpy as jnp
import numpy as np
from jax.experimental import pallas as pl
from jax.experimental.pallas import tpu as pltpu

B = 16
D = 1024
VOCAB = 100000
NEG = -1e9
BN = 3072
NBLK = (VOCAB + BN - 1) // BN
VPAD = NBLK * BN

_KEY_LO = np.int32(np.int64(-2147483648)
                   - np.int64(np.float32(-3.0e38).view(np.int32))
                   - 1)


def _f32_to_key(x):
    b = jax.lax.bitcast_convert_type(x, jnp.int32)
    return jnp.where(b >= 0, b, jnp.int32(-2147483648) - b - 1)


def _key_to_f32(k):
    b = jnp.where(k >= 0, k, jnp.int32(-2147483648) - k - 1)
    return jax.lax.bitcast_convert_type(b, jnp.float32)


def _fused_kernel(hidden_ref, emb_ref, temp_ref, tp_ref, tk_ref,
                  arg_ref, probs_ref, logp_ref,
                  v_s, m_s, z_s, arg_s):
    j = pl.program_id(0)

    @pl.when(j == 0)
    def _init():
        m_s[...] = jnp.full_like(m_s, -jnp.inf)
        z_s[...] = jnp.zeros_like(z_s)
        arg_s[...] = jnp.zeros_like(arg_s)

    logits = jax.lax.dot_general(
        hidden_ref[...], emb_ref[...], (((1,), (1,)), ((), ())),
        preferred_element_type=jnp.float32)
    logits = logits / temp_ref[...]

    col = j * BN + jax.lax.broadcasted_iota(jnp.int32, (B, BN), 1)
    lw = jnp.where(col < VOCAB, logits, -jnp.inf)
    v_s[:, pl.ds(j * BN, BN)] = lw

    bm = jnp.max(lw, axis=1, keepdims=True)
    barg = jnp.min(jnp.where(lw == bm, col, jnp.int32(2147483647)),
                   axis=1, keepdims=True)

    m_old = m_s[...]
    m_new = jnp.maximum(m_old, bm)
    z_s[...] = (z_s[...] * jnp.exp(m_old - m_new)
                + jnp.sum(jnp.exp(lw - m_new), axis=1, keepdims=True))
    arg_s[...] = jnp.where(bm > m_old, barg, arg_s[...])
    m_s[...] = m_new

    @pl.when(j == NBLK - 1)
    def _select_emit():
        m = m_s[...]
        probs_ref[...] = jnp.exp(v_s[:, :VOCAB] - m)

        budget = tp_ref[...] * z_s[...]
        topk = tk_ref[...]

        lo0 = jnp.full((B, 1), _KEY_LO, jnp.int32)
        hi0 = _f32_to_key(m)

        def body(_, carry):
            lo, hi = carry
            mid = (lo >> 1) + (hi >> 1) + (lo & hi & 1)
            tau = _key_to_f32(mid)
            mask = v_s[:, :VOCAB] > tau
            cnt = jnp.sum(jnp.where(mask, 1.0, 0.0), axis=1, keepdims=True)
            mass = jnp.sum(jnp.where(mask, probs_ref[...], 0.0), axis=1,
                           keepdims=True)
            good = (mass <= budget) & (cnt < topk)
            lo = jnp.where(good, lo, mid)
            hi = jnp.where(good, mid, hi)
            return lo, hi

        lo, _ = jax.lax.fori_loop(0, 32, body, (lo0, hi0))
        t_lo = _key_to_f32(lo)

        v = v_s[:, :VOCAB]
        keep = v > t_lo
        ek = jnp.where(keep, probs_ref[...], 0.0)
        zk = jnp.sum(ek, axis=1, keepdims=True)
        arg_ref[...] = arg_s[...]
        probs_ref[...] = ek * (1.0 / zk)
        logp_ref[...] = (jnp.where(keep, v, NEG) - m) - jnp.log(zk)


@jax.jit
def kernel(hidden_states, embedding, temperatures, top_ps, top_ks):
    temp = temperatures.reshape(B, 1)
    tp = top_ps.reshape(B, 1)
    tk = top_ks.astype(jnp.float32).reshape(B, 1)

    arg, probs, logp = pl.pallas_call(
        _fused_kernel,
        grid=(NBLK,),
        in_specs=[
            pl.BlockSpec((B, D), lambda j: (0, 0)),
            pl.BlockSpec((BN, D), lambda j: (j, 0)),
            pl.BlockSpec((B, 1), lambda j: (0, 0)),
            pl.BlockSpec((B, 1), lambda j: (0, 0)),
            pl.BlockSpec((B, 1), lambda j: (0, 0)),
        ],
        out_specs=[
            pl.BlockSpec((B, 1), lambda j: (0, 0)),
            pl.BlockSpec((B, VOCAB), lambda j: (0, 0)),
            pl.BlockSpec((B, VOCAB), lambda j: (0, 0)),
        ],
        out_shape=[
            jax.ShapeDtypeStruct((B, 1), jnp.int32),
            jax.ShapeDtypeStruct((B, VOCAB), jnp.float32),
            jax.ShapeDtypeStruct((B, VOCAB), jnp.float32),
        ],
        scratch_shapes=[
            pltpu.VMEM((B, VPAD), jnp.float32),
            pltpu.VMEM((B, 1), jnp.float32),
            pltpu.VMEM((B, 1), jnp.float32),
            pltpu.VMEM((B, 1), jnp.int32),
        ],
        compiler_params=pltpu.CompilerParams(
            vmem_limit_bytes=100 * 1024 * 1024),
    )(hidden_states, embedding, temp, tp, tk)

    return arg[:, 0], probs, logp
